# baseline (speedup 1.0000x reference)
import functools

import jax
import jax.numpy as jnp
from jax import lax
from jax.experimental import pallas as pl
from jax.experimental.pallas import tpu as pltpu
from jax.experimental.pallas import tpu_sc as plsc

N = 10000
E = 320000
D = 128
ED = 16

NC = 2
NS = 16
NW = NC * NS
CHUNK = 128
NCHUNKS = E // CHUNK
SG = 16
CPT = 80
NCH_PAD = NW * CPT
E_PAD = NCH_PAD * CHUNK
NBUF = 2
NQ = 1
QR = CHUNK // NQ
ROWS_PER_TILE = 632
N_PAD = NS * ROWS_PER_TILE


def _tc_prologue(x, ea_r, W_fc, b_fc, Ws, Wt, Wd, b_att):
    XB = 1000
    E8 = E // 8
    EB = E8 // (N // XB)

    def body(x_ref, ea_ref, wfc_ref, bfc_ref, ws_ref, wt_ref, wd_ref, ba_ref,
             z_ref, s_ref, t_ref, self_ref, u_ref):
        b = ba_ref[0, 0]
        z = jnp.dot(x_ref[...], wfc_ref[...],
                    preferred_element_type=jnp.float32) + bfc_ref[...]
        s = jnp.dot(z, ws_ref[...], preferred_element_type=jnp.float32)
        t = jnp.dot(z, wt_ref[...], preferred_element_type=jnp.float32)
        a = s + t + b
        e_loop = jnp.maximum(a, 0.2 * a)
        z_ref[...] = z
        s_ref[...] = s
        t_ref[...] = t
        self_ref[...] = (1.0 + e_loop) * z
        u = jnp.dot(ea_ref[...], wd_ref[...],
                    preferred_element_type=jnp.float32) + b
        u_ref[...] = u

    grid = N // XB
    out = pl.pallas_call(
        body,
        grid=(grid,),
        in_specs=[
            pl.BlockSpec((XB, D), lambda i: (i, 0)),
            pl.BlockSpec((EB, D), lambda i: (i, 0)),
            pl.BlockSpec((D, D), lambda i: (0, 0)),
            pl.BlockSpec((1, D), lambda i: (0, 0)),
            pl.BlockSpec((D, 1), lambda i: (0, 0)),
            pl.BlockSpec((D, 1), lambda i: (0, 0)),
            pl.BlockSpec((D, 8), lambda i: (0, 0)),
            pl.BlockSpec((1, 1), lambda i: (0, 0)),
        ],
        out_specs=[
            pl.BlockSpec((XB, D), lambda i: (i, 0)),
            pl.BlockSpec((XB, 1), lambda i: (i, 0)),
            pl.BlockSpec((XB, 1), lambda i: (i, 0)),
            pl.BlockSpec((XB, D), lambda i: (i, 0)),
            pl.BlockSpec((EB, 8), lambda i: (i, 0)),
        ],
        out_shape=[
            jax.ShapeDtypeStruct((N, D), jnp.float32),
            jax.ShapeDtypeStruct((N, 1), jnp.float32),
            jax.ShapeDtypeStruct((N, 1), jnp.float32),
            jax.ShapeDtypeStruct((N, D), jnp.float32),
            jax.ShapeDtypeStruct((E_PAD // 8, 8), jnp.float32),
        ],
    )(x, ea_r, W_fc, b_fc.reshape(1, D), Ws, Wt, Wd,
      b_att.reshape(1, 1))
    return out


def _tc_index_prep(ei_r):
    RB = 640

    def body(e_ref, o_ref):
        o_ref[...] = e_ref[...]

    return pl.pallas_call(
        body,
        grid=(NCH_PAD // RB,),
        in_specs=[pl.BlockSpec((2, RB, CHUNK), lambda i: (0, i, 0))],
        out_specs=pl.BlockSpec((2, RB, CHUNK), lambda i: (0, i, 0)),
        out_shape=jax.ShapeDtypeStruct((2, NCH_PAD, CHUNK), jnp.int32),
    )(ei_r)


def _sc_scatter(z, s, t, u3, rc4):
    mesh = plsc.VectorSubcoreMesh(core_axis_name="c", subcore_axis_name="s")

    @functools.partial(
        pl.kernel,
        mesh=mesh,
        out_type=jax.ShapeDtypeStruct((NC, N_PAD, D), jnp.float32),
        scratch_types=[
            pltpu.VMEM_SHARED((N_PAD, D), jnp.float32),
            pltpu.VMEM((2, SG, CHUNK), jnp.int32),
            pltpu.VMEM((2, SG, CHUNK), jnp.int32),
            pltpu.VMEM((2, SG, CHUNK), jnp.float32),
            pltpu.VMEM((NBUF, CHUNK, D), jnp.float32),
            pltpu.VMEM((NBUF, CHUNK), jnp.float32),
            pltpu.VMEM((NBUF, CHUNK), jnp.float32),
            pltpu.SemaphoreType.DMA,
            pltpu.SemaphoreType.DMA,
            pltpu.SemaphoreType.DMA,
            pltpu.SemaphoreType.DMA,
            pltpu.SemaphoreType.DMA,
        ],
    )
    def k(z_hbm, s_hbm, t_hbm, u_hbm, rc_hbm, out_hbm,
          hacc, row_v, col_v, u_v, zrows, sgb, tgb,
          g0, g1, p0, p1, sem_meta):
        c = lax.axis_index("c")
        sid = lax.axis_index("s")
        wid = c * NS + sid
        gsems = (g0, g1)
        psems = (p0, p1)
        zb = tuple(zrows.at[i] for i in range(NBUF))
        sb = tuple(sgb.at[i] for i in range(NBUF))
        tb = tuple(tgb.at[i] for i in range(NBUF))


        def refill(g):
            off = pl.ds(pl.multiple_of(g * SG, SG), SG)
            slot = lax.rem(g, 2)
            pltpu.async_copy(rc_hbm.at[0].at[wid].at[off], row_v.at[slot],
                             sem_meta)
            pltpu.async_copy(rc_hbm.at[1].at[wid].at[off], col_v.at[slot],
                             sem_meta)
            pltpu.async_copy(u_hbm.at[wid].at[off], u_v.at[slot], sem_meta)

        def wait_refill(g):
            slot = lax.rem(g, 2)
            pltpu.make_async_copy(rc_hbm.at[0].at[wid].at[pl.ds(0, SG)],
                                  row_v.at[slot], sem_meta).wait()
            pltpu.make_async_copy(rc_hbm.at[1].at[wid].at[pl.ds(0, SG)],
                                  col_v.at[slot], sem_meta).wait()
            pltpu.make_async_copy(u_hbm.at[wid].at[pl.ds(0, SG)],
                                  u_v.at[slot], sem_meta).wait()

        refill(0)

        @plsc.parallel_loop(0, CHUNK, step=1, unroll=4)
        def _(i):
            for d in range(D // 16):
                zb[0][i, pl.ds(d * 16, 16)] = jnp.zeros((16,), jnp.float32)

        base = sid * ROWS_PER_TILE
        for kk in range(ROWS_PER_TILE // CHUNK):
            pltpu.sync_copy(zb[0], hacc.at[pl.ds(base + kk * CHUNK, CHUNK)])
        rem = ROWS_PER_TILE % CHUNK
        if rem:
            pltpu.sync_copy(
                zb[0].at[pl.ds(0, rem)],
                hacc.at[pl.ds(base + (ROWS_PER_TILE // CHUNK) * CHUNK, rem)])

        plsc.subcore_barrier()

        nch = jnp.minimum(CPT, jnp.maximum(0, NCHUNKS - wid * CPT))

        def _idx(j):
            g = lax.div(j, SG)
            return lax.rem(g, 2), lax.rem(j, SG), g

        def start(j, b):
            slot, local, g = _idx(j)

            @pl.when(local == 0)
            def _():
                wait_refill(g)

            @pl.when(j >= NBUF)
            def _():
                for q in range(NQ):
                    pltpu.make_async_copy(
                        zb[b].at[pl.ds(q * QR, QR)],
                        hacc.at[col_v.at[slot].at[local].at[pl.ds(q * QR, QR)]],
                        psems[b]).wait()
            ridx = row_v.at[slot].at[local]
            cidx = col_v.at[slot].at[local]
            pltpu.async_copy(z_hbm.at[ridx], zb[b], gsems[b])
            pltpu.async_copy(s_hbm.at[ridx], sb[b], gsems[b])
            pltpu.async_copy(t_hbm.at[cidx], tb[b], gsems[b])

        def finish(j, b):
            slot, local, _ = _idx(j)
            ridx = row_v.at[slot].at[local]
            cidx = col_v.at[slot].at[local]
            pltpu.make_async_copy(z_hbm.at[ridx], zb[b], gsems[b]).wait()
            pltpu.make_async_copy(s_hbm.at[ridx], sb[b], gsems[b]).wait()
            pltpu.make_async_copy(t_hbm.at[cidx], tb[b], gsems[b]).wait()

        def compute(j, b):
            slot, local, g = _idx(j)

            @pl.when((local == 0) & ((g + 1) * SG < nch))
            def _():
                refill(g + 1)

            for q in range(NQ):
                @plsc.parallel_loop(0, QR // 16, step=1, unroll=2)
                def _(gg, q=q):
                    g8 = q * (QR // 16) + gg
                    sl = pl.ds(pl.multiple_of(g8 * 16, 16), 16)
                    a = sb[b][sl] + tb[b][sl] + u_v[slot, local, sl]
                    w16 = jnp.maximum(a, 0.2 * a)
                    for l in range(16):
                        wv = jnp.take(w16, jnp.full((16,), l, jnp.int32),
                                      axis=0)
                        for d in range(D // 16):
                            dsl = pl.ds(d * 16, 16)
                            e = g8 * 16 + l
                            zb[b][e, dsl] = zb[b][e, dsl] * wv

                pltpu.async_copy(
                    zb[b].at[pl.ds(q * QR, QR)],
                    hacc.at[col_v.at[slot].at[local].at[pl.ds(q * QR, QR)]],
                    psems[b], add=True)

        start(0, 0)

        def body(j2, carry):
            for b in range(NBUF):
                j = NBUF * j2 + b

                @pl.when(j < nch)
                def _():
                    @pl.when(j + 1 < nch)
                    def _():
                        start(j + 1, (b + 1) % NBUF)
                    finish(j, b)
                    compute(j, b)
            return carry
        lax.fori_loop(0, (nch + NBUF - 1) // NBUF, body, 0)

        for b in range(NBUF):
            jb = nch - 1 - lax.rem(nch - 1 - b + NBUF, NBUF)

            @pl.when(jb >= 0)
            def _(jb=jb, b=b):
                slot, local, _ = _idx(jb)
                for q in range(NQ):
                    pltpu.make_async_copy(
                        zrows.at[b].at[pl.ds(q * QR, QR)],
                        hacc.at[col_v.at[slot].at[local].at[pl.ds(q * QR, QR)]],
                        psems[b]).wait()

        plsc.subcore_barrier()
        pltpu.sync_copy(hacc.at[pl.ds(base, ROWS_PER_TILE)],
                        out_hbm.at[c].at[pl.ds(base, ROWS_PER_TILE)])

    return k(z, s, t, u3, rc4)


def _tc_epilogue(hpart, selfterm):
    XB = 1000

    def body(a_ref, b_ref, c_ref, o_ref):
        o_ref[...] = a_ref[0] + b_ref[0] + c_ref[...]

    return pl.pallas_call(
        body,
        grid=(N // XB,),
        in_specs=[
            pl.BlockSpec((1, XB, D), lambda i: (0, i, 0)),
            pl.BlockSpec((1, XB, D), lambda i: (1, i, 0)),
            pl.BlockSpec((XB, D), lambda i: (i, 0)),
        ],
        out_specs=pl.BlockSpec((XB, D), lambda i: (i, 0)),
        out_shape=jax.ShapeDtypeStruct((N, D), jnp.float32),
    )(hpart, hpart, selfterm)


def kernel(x, edge_index, edge_attr, W_fc, b_fc, W_att, b_att):
    Ws = W_att[:D]
    Wt = W_att[D:2 * D]
    Wu = W_att[2 * D:]
    Wd = jnp.kron(jnp.eye(8, dtype=jnp.float32), Wu)
    ea_r = edge_attr.reshape(E // 8, 8 * ED)

    z, s, t, selfterm, u = _tc_prologue(x, ea_r, W_fc, b_fc, Ws, Wt, Wd,
                                        b_att)

    rc = _tc_index_prep(edge_index.reshape(2, NCHUNKS, CHUNK))
    rc4 = rc.reshape(2, NW, CPT, CHUNK)
    u3 = u.reshape(NW, CPT, CHUNK)

    hpart = _sc_scatter(z, s.reshape(N), t.reshape(N), u3, rc4)

    return _tc_epilogue(hpart, selfterm)


# speedup vs baseline: 1.0312x; 1.0312x over previous
import functools

import jax
import jax.numpy as jnp
from jax import lax
from jax.experimental import pallas as pl
from jax.experimental.pallas import tpu as pltpu
from jax.experimental.pallas import tpu_sc as plsc

N = 10000
E = 320000
D = 128
ED = 16

NC = 2
NS = 16
NW = NC * NS
CHUNK = 128
NCHUNKS = E // CHUNK
SG = 8
CPT = 80
NCH_PAD = NW * CPT
E_PAD = NCH_PAD * CHUNK
NBUF = 2
NQ = 2
QR = CHUNK // NQ
ROWS_PER_TILE = 632
N_PAD = NS * ROWS_PER_TILE


def _tc_prologue(x, ea_r, W_fc, b_fc, Ws, Wt, Wd, b_att):
    XB = 1000
    E8 = E // 8
    EB = E8 // (N // XB)

    def body(x_ref, ea_ref, wfc_ref, bfc_ref, ws_ref, wt_ref, wd_ref, ba_ref,
             z_ref, s_ref, t_ref, self_ref, u_ref):
        b = ba_ref[0, 0]
        z = jnp.dot(x_ref[...], wfc_ref[...],
                    preferred_element_type=jnp.float32) + bfc_ref[...]
        s = jnp.dot(z, ws_ref[...], preferred_element_type=jnp.float32)
        t = jnp.dot(z, wt_ref[...], preferred_element_type=jnp.float32)
        a = s + t + b
        e_loop = jnp.maximum(a, 0.2 * a)
        z_ref[...] = z
        s_ref[...] = s
        t_ref[...] = t
        self_ref[...] = (1.0 + e_loop) * z
        u = jnp.dot(ea_ref[...], wd_ref[...],
                    preferred_element_type=jnp.float32) + b
        u_ref[...] = u

    grid = N // XB
    out = pl.pallas_call(
        body,
        grid=(grid,),
        in_specs=[
            pl.BlockSpec((XB, D), lambda i: (i, 0)),
            pl.BlockSpec((EB, D), lambda i: (i, 0)),
            pl.BlockSpec((D, D), lambda i: (0, 0)),
            pl.BlockSpec((1, D), lambda i: (0, 0)),
            pl.BlockSpec((D, 1), lambda i: (0, 0)),
            pl.BlockSpec((D, 1), lambda i: (0, 0)),
            pl.BlockSpec((D, 8), lambda i: (0, 0)),
            pl.BlockSpec((1, 1), lambda i: (0, 0)),
        ],
        out_specs=[
            pl.BlockSpec((XB, D), lambda i: (i, 0)),
            pl.BlockSpec((XB, 1), lambda i: (i, 0)),
            pl.BlockSpec((XB, 1), lambda i: (i, 0)),
            pl.BlockSpec((XB, D), lambda i: (i, 0)),
            pl.BlockSpec((EB, 8), lambda i: (i, 0)),
        ],
        out_shape=[
            jax.ShapeDtypeStruct((N, D), jnp.float32),
            jax.ShapeDtypeStruct((N, 1), jnp.float32),
            jax.ShapeDtypeStruct((N, 1), jnp.float32),
            jax.ShapeDtypeStruct((N, D), jnp.float32),
            jax.ShapeDtypeStruct((E_PAD // 8, 8), jnp.float32),
        ],
    )(x, ea_r, W_fc, b_fc.reshape(1, D), Ws, Wt, Wd,
      b_att.reshape(1, 1))
    return out


def _tc_index_prep(ei_r):
    RB = 640

    def body(e_ref, o_ref):
        o_ref[...] = e_ref[...]

    return pl.pallas_call(
        body,
        grid=(NCH_PAD // RB,),
        in_specs=[pl.BlockSpec((2, RB, CHUNK), lambda i: (0, i, 0))],
        out_specs=pl.BlockSpec((2, RB, CHUNK), lambda i: (0, i, 0)),
        out_shape=jax.ShapeDtypeStruct((2, NCH_PAD, CHUNK), jnp.int32),
    )(ei_r)


def _sc_scatter(z, s, t, u3, rc4):
    mesh = plsc.VectorSubcoreMesh(core_axis_name="c", subcore_axis_name="s")

    @functools.partial(
        pl.kernel,
        mesh=mesh,
        out_type=jax.ShapeDtypeStruct((NC, N_PAD, D), jnp.float32),
        scratch_types=[
            pltpu.VMEM_SHARED((N_PAD, D), jnp.float32),
            pltpu.VMEM((2, SG, CHUNK), jnp.int32),
            pltpu.VMEM((2, SG, CHUNK), jnp.int32),
            pltpu.VMEM((2, SG, CHUNK), jnp.float32),
            pltpu.VMEM((NBUF, CHUNK, D), jnp.float32),
            pltpu.VMEM((2, SG, CHUNK), jnp.float32),
            pltpu.VMEM((2, SG, CHUNK), jnp.float32),
            pltpu.SemaphoreType.DMA,
            pltpu.SemaphoreType.DMA,
            pltpu.SemaphoreType.DMA,
            pltpu.SemaphoreType.DMA,
            pltpu.SemaphoreType.DMA,
            pltpu.SemaphoreType.DMA,
        ],
    )
    def k(z_hbm, s_hbm, t_hbm, u_hbm, rc_hbm, out_hbm,
          hacc, row_v, col_v, u_v, zrows, sgb, tgb,
          g0, g1, p0, p1, sem_meta, sem_sg):
        c = lax.axis_index("c")
        sid = lax.axis_index("s")
        wid = c * NS + sid
        gsems = (g0, g1)
        psems = (p0, p1)
        zb = tuple(zrows.at[i] for i in range(NBUF))


        def refill(g):
            off = pl.ds(pl.multiple_of(g * SG, SG), SG)
            slot = lax.rem(g, 2)
            pltpu.async_copy(rc_hbm.at[0].at[wid].at[off], row_v.at[slot],
                             sem_meta)
            pltpu.async_copy(rc_hbm.at[1].at[wid].at[off], col_v.at[slot],
                             sem_meta)
            pltpu.async_copy(u_hbm.at[wid].at[off], u_v.at[slot], sem_meta)

        def wait_refill(g):
            slot = lax.rem(g, 2)
            pltpu.make_async_copy(rc_hbm.at[0].at[wid].at[pl.ds(0, SG)],
                                  row_v.at[slot], sem_meta).wait()
            pltpu.make_async_copy(rc_hbm.at[1].at[wid].at[pl.ds(0, SG)],
                                  col_v.at[slot], sem_meta).wait()
            pltpu.make_async_copy(u_hbm.at[wid].at[pl.ds(0, SG)],
                                  u_v.at[slot], sem_meta).wait()

        refill(0)

        @plsc.parallel_loop(0, CHUNK, step=1, unroll=4)
        def _(i):
            for d in range(D // 16):
                zb[0][i, pl.ds(d * 16, 16)] = jnp.zeros((16,), jnp.float32)

        base = sid * ROWS_PER_TILE
        for kk in range(ROWS_PER_TILE // CHUNK):
            pltpu.sync_copy(zb[0], hacc.at[pl.ds(base + kk * CHUNK, CHUNK)])
        rem = ROWS_PER_TILE % CHUNK
        if rem:
            pltpu.sync_copy(
                zb[0].at[pl.ds(0, rem)],
                hacc.at[pl.ds(base + (ROWS_PER_TILE // CHUNK) * CHUNK, rem)])

        plsc.subcore_barrier()

        nch = jnp.minimum(CPT, jnp.maximum(0, NCHUNKS - wid * CPT))

        def _idx(j):
            g = lax.div(j, SG)
            return lax.rem(g, 2), lax.rem(j, SG), g

        def start(j, b):
            slot, local, g = _idx(j)

            @pl.when(local == 0)
            def _():
                wait_refill(g)

            pltpu.async_copy(s_hbm.at[row_v.at[slot].at[local]],
                             sgb.at[slot].at[local], sem_sg)
            pltpu.async_copy(t_hbm.at[col_v.at[slot].at[local]],
                             tgb.at[slot].at[local], sem_sg)

            @pl.when(j >= NBUF)
            def _():
                for q in range(NQ):
                    pltpu.make_async_copy(
                        zb[b].at[pl.ds(q * QR, QR)],
                        hacc.at[col_v.at[slot].at[local].at[pl.ds(q * QR, QR)]],
                        psems[b]).wait()
            ridx = row_v.at[slot].at[local]
            cidx = col_v.at[slot].at[local]
            pltpu.async_copy(z_hbm.at[ridx], zb[b], gsems[b])

        def finish(j, b):
            slot, local, _ = _idx(j)
            ridx = row_v.at[slot].at[local]
            pltpu.make_async_copy(z_hbm.at[ridx], zb[b], gsems[b]).wait()

        def compute(j, b):
            slot, local, g = _idx(j)

            @pl.when((local == 0) & ((g + 1) * SG < nch))
            def _():
                refill(g + 1)

            pltpu.make_async_copy(s_hbm.at[row_v.at[slot].at[local]],
                                  sgb.at[slot].at[local], sem_sg).wait()
            pltpu.make_async_copy(t_hbm.at[col_v.at[slot].at[local]],
                                  tgb.at[slot].at[local], sem_sg).wait()

            for q in range(NQ):
                @plsc.parallel_loop(0, QR // 16, step=1, unroll=2)
                def _(gg, q=q):
                    g8 = q * (QR // 16) + gg
                    sl = pl.ds(pl.multiple_of(g8 * 16, 16), 16)
                    a = (sgb[slot, local, sl] + tgb[slot, local, sl]
                         + u_v[slot, local, sl])
                    w16 = jnp.maximum(a, 0.2 * a)
                    for l in range(16):
                        wv = jnp.take(w16, jnp.full((16,), l, jnp.int32),
                                      axis=0)
                        for d in range(D // 16):
                            dsl = pl.ds(d * 16, 16)
                            e = g8 * 16 + l
                            zb[b][e, dsl] = zb[b][e, dsl] * wv

                pltpu.async_copy(
                    zb[b].at[pl.ds(q * QR, QR)],
                    hacc.at[col_v.at[slot].at[local].at[pl.ds(q * QR, QR)]],
                    psems[b], add=True)

        start(0, 0)

        def body(j2, carry):
            for b in range(NBUF):
                j = NBUF * j2 + b

                @pl.when(j < nch)
                def _():
                    @pl.when(j + 1 < nch)
                    def _():
                        start(j + 1, (b + 1) % NBUF)
                    finish(j, b)
                    compute(j, b)
            return carry
        lax.fori_loop(0, (nch + NBUF - 1) // NBUF, body, 0)

        for b in range(NBUF):
            jb = nch - 1 - lax.rem(nch - 1 - b + NBUF, NBUF)

            @pl.when(jb >= 0)
            def _(jb=jb, b=b):
                slot, local, _ = _idx(jb)
                for q in range(NQ):
                    pltpu.make_async_copy(
                        zrows.at[b].at[pl.ds(q * QR, QR)],
                        hacc.at[col_v.at[slot].at[local].at[pl.ds(q * QR, QR)]],
                        psems[b]).wait()

        plsc.subcore_barrier()
        pltpu.sync_copy(hacc.at[pl.ds(base, ROWS_PER_TILE)],
                        out_hbm.at[c].at[pl.ds(base, ROWS_PER_TILE)])

    return k(z, s, t, u3, rc4)


def _tc_epilogue(hpart, selfterm):
    XB = 1000

    def body(a_ref, b_ref, c_ref, o_ref):
        o_ref[...] = a_ref[0] + b_ref[0] + c_ref[...]

    return pl.pallas_call(
        body,
        grid=(N // XB,),
        in_specs=[
            pl.BlockSpec((1, XB, D), lambda i: (0, i, 0)),
            pl.BlockSpec((1, XB, D), lambda i: (1, i, 0)),
            pl.BlockSpec((XB, D), lambda i: (i, 0)),
        ],
        out_specs=pl.BlockSpec((XB, D), lambda i: (i, 0)),
        out_shape=jax.ShapeDtypeStruct((N, D), jnp.float32),
    )(hpart, hpart, selfterm)


def kernel(x, edge_index, edge_attr, W_fc, b_fc, W_att, b_att):
    Ws = W_att[:D]
    Wt = W_att[D:2 * D]
    Wu = W_att[2 * D:]
    Wd = jnp.kron(jnp.eye(8, dtype=jnp.float32), Wu)
    ea_r = edge_attr.reshape(E // 8, 8 * ED)

    z, s, t, selfterm, u = _tc_prologue(x, ea_r, W_fc, b_fc, Ws, Wt, Wd,
                                        b_att)

    rc = _tc_index_prep(edge_index.reshape(2, NCHUNKS, CHUNK))
    rc4 = rc.reshape(2, NW, CPT, CHUNK)
    u3 = u.reshape(NW, CPT, CHUNK)

    hpart = _sc_scatter(z, s.reshape(N), t.reshape(N), u3, rc4)

    return _tc_epilogue(hpart, selfterm)
